# baseline (speedup 1.0000x reference)
import functools

import jax
import jax.numpy as jnp
from jax import lax
from jax.experimental import pallas as pl
from jax.experimental.pallas import tpu as pltpu
from jax.experimental.pallas import tpu_sc as plsc

_C = 19
_N = 4194304
_NC = 2
_NS = 16
_NW = _NC * _NS
_ROWS_PER_TILE = _N // _NW
_CH = 2048
_NCHUNK = _ROWS_PER_TILE // _CH
_BSTRIDE = 32
_NBINS = _C * _BSTRIDE


def _sc_partial_cm(labels, predictions):
    mesh = plsc.VectorSubcoreMesh(core_axis_name="c", subcore_axis_name="s")

    @functools.partial(
        pl.kernel,
        mesh=mesh,
        out_type=jax.ShapeDtypeStruct((_NW, _C, _BSTRIDE), jnp.float32),
        compiler_params=pltpu.CompilerParams(
            needs_layout_passes=False, use_tc_tiling_on_sc=False),
        scratch_types=[
            pltpu.VMEM((_CH, _C), jnp.float32),
            pltpu.VMEM((_CH,), jnp.int32),
            pltpu.VMEM((16, _NBINS), jnp.float32),
            pltpu.VMEM((_C, _BSTRIDE), jnp.float32),
        ],
    )
    def k(lab_hbm, pred_hbm, out_hbm, pbuf, lbuf, bins, res):
        wid = lax.axis_index("s") * _NC + lax.axis_index("c")
        base0 = wid * _ROWS_PER_TILE
        iota16 = lax.iota(jnp.int32, 16)
        ones16 = jnp.full((16,), 1.0, jnp.float32)
        zeros16 = jnp.zeros((16,), jnp.float32)

        def zero_row(r, carry):
            for j in range(_NBINS // 16):
                bins[r, pl.ds(j * 16, 16)] = zeros16
            return carry

        lax.fori_loop(0, 16, zero_row, 0)

        def chunk_body(kk, carry):
            base = base0 + kk * _CH
            pltpu.sync_copy(pred_hbm.at[pl.ds(base, _CH)], pbuf)
            pltpu.sync_copy(lab_hbm.at[pl.ds(base, _CH)], lbuf)

            def row_body(i, c2):
                rid = jnp.full((16,), i * 16, jnp.int32) + iota16
                cols = [
                    plsc.load_gather(
                        pbuf, [rid, jnp.full((16,), c, jnp.int32)])
                    for c in range(_C)
                ]
                m = cols[_C - 1]
                idx = jnp.full((16,), _C - 1, jnp.int32)
                for c in range(_C - 2, -1, -1):
                    take = cols[c] >= m
                    m = jnp.where(take, cols[c], m)
                    idx = jnp.where(take, jnp.full((16,), c, jnp.int32), idx)
                labv = lbuf[pl.ds(i * 16, 16)]
                flat = labv * _BSTRIDE + idx
                plsc.addupdate_scatter(bins, [iota16, flat], ones16)
                return c2

            lax.fori_loop(0, _CH // 16, row_body, 0)
            return carry

        lax.fori_loop(0, _NCHUNK, chunk_body, 0)

        for row in range(_C):
            for h in range(_BSTRIDE // 16):
                acc = bins[0, pl.ds(row * _BSTRIDE + h * 16, 16)]
                for r in range(1, 16):
                    acc = acc + bins[r, pl.ds(row * _BSTRIDE + h * 16, 16)]
                res[row, pl.ds(h * 16, 16)] = acc
        pltpu.sync_copy(res, out_hbm.at[wid])

    return k(labels, predictions)


def _finalize_kernel(part_ref, cm_ref, stats_ref):
    x = part_ref[...]
    t = jnp.sum(x, axis=0)
    cm = t[:, :_C]
    c = _C
    r = lax.broadcasted_iota(jnp.int32, (c, c), 0)
    q = lax.broadcasted_iota(jnp.int32, (c, c), 1)
    eye = (r == q).astype(jnp.float32)
    ones = jnp.ones((1, c), jnp.float32)
    tps = jnp.sum(cm * eye, axis=0, keepdims=True)
    colsum = lax.dot_general(
        ones, cm, (((1,), (0,)), ((), ())),
        preferred_element_type=jnp.float32)
    rowsum = lax.dot_general(
        ones, cm, (((1,), (1,)), ((), ())),
        preferred_element_type=jnp.float32)
    fps = colsum - tps
    fns = rowsum - tps
    precisions = tps / (tps + fps)
    recalls = tps / (tps + fns)
    ious = tps / (tps + fps + fns)
    cm_ref[...] = cm
    stats_ref[...] = jnp.concatenate(
        [tps, fps, fns, precisions, recalls, ious], axis=0)


def kernel(labels, predictions):
    part = _sc_partial_cm(labels, predictions)
    cm, stats = pl.pallas_call(
        _finalize_kernel,
        out_shape=[
            jax.ShapeDtypeStruct((_C, _C), jnp.float32),
            jax.ShapeDtypeStruct((6, _C), jnp.float32),
        ],
    )(part)
    return (cm, stats[0], stats[1], stats[2], stats[3], stats[4], stats[5])


# speedup vs baseline: 2.0147x; 2.0147x over previous
import jax
import jax.numpy as jnp
from jax.experimental import pallas as pl

_C = 19
_N = 4194304
_B = 32768
_SPLIT = 2


def _half_cm_kernel(lab_ref, pred_ref, cm_ref):
    i = pl.program_id(0)

    @pl.when(i == 0)
    def _init():
        cm_ref[...] = jnp.zeros_like(cm_ref)

    p = pred_ref[...]
    b, c = p.shape
    pt = jnp.transpose(p)
    m = jnp.max(pt, axis=0, keepdims=True)
    srow = jax.lax.broadcasted_iota(jnp.int32, (c, b), 0)
    idx = jnp.min(jnp.where(pt == m, srow, c), axis=0, keepdims=True)
    pred_oh = (srow == idx).astype(jnp.bfloat16)
    lab = lab_ref[0, 0, :]
    lab_oh = (lab[None, :] == srow).astype(jnp.bfloat16)
    cm_ref[...] += jax.lax.dot_general(
        lab_oh, pred_oh, (((1,), (1,)), ((), ())),
        preferred_element_type=jnp.float32)


def _half_cm(labels, predictions):
    n, c = predictions.shape
    nb = n // _B
    lab3 = labels.reshape(nb, 1, _B)
    return pl.pallas_call(
        _half_cm_kernel,
        grid=(nb,),
        in_specs=[
            pl.BlockSpec((1, 1, _B), lambda i: (i, 0, 0)),
            pl.BlockSpec((_B, c), lambda i: (i, 0)),
        ],
        out_specs=pl.BlockSpec((c, c), lambda i: (0, 0)),
        out_shape=jax.ShapeDtypeStruct((c, c), jnp.float32),
    )(lab3, predictions)


def _finalize_kernel(cm1_ref, cm2_ref, cm_ref, stats_ref):
    cm = cm1_ref[...] + cm2_ref[...]
    c = _C
    r = jax.lax.broadcasted_iota(jnp.int32, (c, c), 0)
    q = jax.lax.broadcasted_iota(jnp.int32, (c, c), 1)
    eye = (r == q).astype(jnp.float32)
    ones = jnp.ones((1, c), jnp.float32)
    tps = jnp.sum(cm * eye, axis=0, keepdims=True)
    colsum = jax.lax.dot_general(
        ones, cm, (((1,), (0,)), ((), ())),
        preferred_element_type=jnp.float32)
    rowsum = jax.lax.dot_general(
        ones, cm, (((1,), (1,)), ((), ())),
        preferred_element_type=jnp.float32)
    fps = colsum - tps
    fns = rowsum - tps
    precisions = tps / (tps + fps)
    recalls = tps / (tps + fns)
    ious = tps / (tps + fps + fns)
    cm_ref[...] = cm
    stats_ref[...] = jnp.concatenate(
        [tps, fps, fns, precisions, recalls, ious], axis=0)


def kernel(labels, predictions):
    n, c = predictions.shape
    h = n // _SPLIT
    cm1 = _half_cm(labels[:h], predictions[:h])
    cm2 = _half_cm(labels[h:], predictions[h:])
    cm, stats = pl.pallas_call(
        _finalize_kernel,
        out_shape=[
            jax.ShapeDtypeStruct((_C, _C), jnp.float32),
            jax.ShapeDtypeStruct((6, _C), jnp.float32),
        ],
    )(cm1, cm2)
    return (cm, stats[0], stats[1], stats[2], stats[3], stats[4], stats[5])


# speedup vs baseline: 2.3159x; 1.1495x over previous
import jax
import jax.numpy as jnp
from jax.experimental import pallas as pl
from jax.experimental.pallas import tpu as pltpu

_C = 19
_N = 4194304
_B = 32768


def _iou_kernel(lab_ref, pred_ref, cm_ref, stats_ref):
    i = pl.program_id(0)

    @pl.when(i == 0)
    def _init():
        cm_ref[...] = jnp.zeros_like(cm_ref)

    p = pred_ref[...]
    b, c = p.shape
    pt = jnp.transpose(p)
    m = jnp.max(pt, axis=0, keepdims=True)
    srow = jax.lax.broadcasted_iota(jnp.int32, (c, b), 0)
    idx = jnp.min(jnp.where(pt == m, srow, c), axis=0, keepdims=True)
    pred_oh = (srow == idx).astype(jnp.bfloat16)
    lab = lab_ref[0, 0, :]
    lab_oh = (lab[None, :] == srow).astype(jnp.bfloat16)
    cm_ref[...] += jax.lax.dot_general(
        lab_oh, pred_oh, (((1,), (1,)), ((), ())),
        preferred_element_type=jnp.float32)

    @pl.when(i == pl.num_programs(0) - 1)
    def _finalize():
        cm = cm_ref[...]
        r = jax.lax.broadcasted_iota(jnp.int32, (c, c), 0)
        q = jax.lax.broadcasted_iota(jnp.int32, (c, c), 1)
        eye = (r == q).astype(jnp.float32)
        ones = jnp.ones((1, c), jnp.float32)
        tps = jnp.sum(cm * eye, axis=0, keepdims=True)
        colsum = jax.lax.dot_general(
            ones, cm, (((1,), (0,)), ((), ())),
            preferred_element_type=jnp.float32)
        rowsum = jax.lax.dot_general(
            ones, cm, (((1,), (1,)), ((), ())),
            preferred_element_type=jnp.float32)
        fps = colsum - tps
        fns = rowsum - tps
        precisions = tps / (tps + fps)
        recalls = tps / (tps + fns)
        ious = tps / (tps + fps + fns)
        stats_ref[...] = jnp.concatenate(
            [tps, fps, fns, precisions, recalls, ious], axis=0)


def kernel(labels, predictions):
    n, c = predictions.shape
    nb = n // _B
    lab3 = labels.reshape(nb, 1, _B)
    cm, stats = pl.pallas_call(
        _iou_kernel,
        grid=(nb,),
        compiler_params=pltpu.CompilerParams(
            allow_input_fusion=[False, True]),
        in_specs=[
            pl.BlockSpec((1, 1, _B), lambda i: (i, 0, 0)),
            pl.BlockSpec((_B, c), lambda i: (i, 0)),
        ],
        out_specs=[
            pl.BlockSpec((c, c), lambda i: (0, 0)),
            pl.BlockSpec((6, c), lambda i: (0, 0)),
        ],
        out_shape=[
            jax.ShapeDtypeStruct((c, c), jnp.float32),
            jax.ShapeDtypeStruct((6, c), jnp.float32),
        ],
    )(lab3, predictions)
    return (cm, stats[0], stats[1], stats[2], stats[3], stats[4], stats[5])
